# baseline (speedup 1.0000x reference)
import functools
import math

import jax
import jax.numpy as jnp
import numpy as np
from jax import lax
from jax.experimental import pallas as pl
from jax.experimental.pallas import tpu as pltpu
from jax.experimental.pallas import tpu_sc as plsc

B = 16384
D = 64
H = 1024
BN_EPS = 1e-5
_BN_INV = float(1.0 / math.sqrt(1.0 + BN_EPS))

_NC, _NS = 2, 16
_NW = _NC * _NS
_BPW = B // _NW
_PCH = 256
_PK = 32

_PERM = np.concatenate([np.arange(0, 64, 2), np.arange(1, 64, 2),
                        np.arange(64, 128, 2), np.arange(65, 128, 2)])


def _gather_body(u_hbm, m_hbm, ut_hbm, mt_hbm, ue_out, me_out,
                 uidx_v, midx_v, urows, mrows, sem_u, sem_m):
    wid = lax.axis_index("s") * _NC + lax.axis_index("c")
    base = wid * _BPW
    pltpu.sync_copy(u_hbm.at[pl.ds(base, _BPW)], uidx_v)
    pltpu.sync_copy(m_hbm.at[pl.ds(base, _BPW)], midx_v)

    for p in range(_BPW // _PCH):
        off = p * _PCH

        def _grp(g, carry):
            uv = uidx_v[pl.ds(off + g * 16, 16)] >> 2
            mv = midx_v[pl.ds(off + g * 16, 16)] >> 2
            for j in range(16):
                r = g * 16 + j
                pltpu.async_copy(ut_hbm.at[uv[j]], urows.at[r], sem_u)
                pltpu.async_copy(mt_hbm.at[mv[j]], mrows.at[r], sem_m)
            return carry

        lax.fori_loop(0, _PCH // 16, _grp, 0)
        pltpu.make_async_copy(ut_hbm.at[pl.ds(0, _PCH)], urows, sem_u).wait()
        pltpu.make_async_copy(mt_hbm.at[pl.ds(0, _PCH)], mrows, sem_m).wait()
        dst = pl.ds(base + off, _PCH)
        pltpu.sync_copy(urows, ue_out.at[dst])
        pltpu.sync_copy(mrows, me_out.at[dst])


@functools.cache
def _build_gather():
    mesh = plsc.VectorSubcoreMesh(core_axis_name="c", subcore_axis_name="s",
                                  num_cores=_NC, num_subcores=_NS)
    return pl.kernel(
        _gather_body,
        mesh=mesh,
        out_type=[jax.ShapeDtypeStruct((B, 4 * _PK), jnp.int32),
                  jax.ShapeDtypeStruct((B, 4 * _PK), jnp.int32)],
        scratch_types=[
            pltpu.VMEM((_BPW,), jnp.int32),
            pltpu.VMEM((_BPW,), jnp.int32),
            pltpu.VMEM((_PCH, 4 * _PK), jnp.int32),
            pltpu.VMEM((_PCH, 4 * _PK), jnp.int32),
            pltpu.SemaphoreType.DMA,
            pltpu.SemaphoreType.DMA,
        ],
    )


_BLK = 1024


def _unpack_quarter(rows, q):
    w01 = jnp.where(q == 0, rows[:, 0:_PK], rows[:, _PK:2 * _PK])
    w23 = jnp.where(q == 2, rows[:, 2 * _PK:3 * _PK], rows[:, 3 * _PK:])
    w = jnp.where(q < 2, w01, w23)
    lo = lax.bitcast_convert_type(w << 16, jnp.float32)
    hi = lax.bitcast_convert_type(w & jnp.int32(-65536), jnp.float32)
    return lo, hi


def _mlp_body(ue_ref, me_ref, u_ref, m_ref, w1_ref, b1_ref, gamma_ref,
              beta_ref, w2_ref, b2_ref, out_ref):
    ulo, uhi = _unpack_quarter(ue_ref[...], u_ref[...] & 3)
    mlo, mhi = _unpack_quarter(me_ref[...], m_ref[...] & 3)
    x = jnp.concatenate([ulo, uhi, mlo, mhi], axis=1)
    h = jnp.dot(x, w1_ref[...], preferred_element_type=jnp.float32)
    h = jnp.maximum(h + b1_ref[...], 0.0)
    h = h * (gamma_ref[...] * _BN_INV) + beta_ref[...]
    out = jnp.sum(h * w2_ref[...], axis=1, keepdims=True) + b2_ref[...]
    out_ref[...] = out


_mlp = pl.pallas_call(
    _mlp_body,
    grid=(B // _BLK,),
    in_specs=[
        pl.BlockSpec((_BLK, 4 * _PK), lambda i: (i, 0)),
        pl.BlockSpec((_BLK, 4 * _PK), lambda i: (i, 0)),
        pl.BlockSpec((_BLK, 1), lambda i: (i, 0)),
        pl.BlockSpec((_BLK, 1), lambda i: (i, 0)),
        pl.BlockSpec((2 * D, H), lambda i: (0, 0)),
        pl.BlockSpec((1, H), lambda i: (0, 0)),
        pl.BlockSpec((1, H), lambda i: (0, 0)),
        pl.BlockSpec((1, H), lambda i: (0, 0)),
        pl.BlockSpec((1, H), lambda i: (0, 0)),
        pl.BlockSpec((1, 1), lambda i: (0, 0)),
    ],
    out_specs=pl.BlockSpec((_BLK, 1), lambda i: (i, 0)),
    out_shape=jax.ShapeDtypeStruct((B, 1), jnp.float32),
)


def _pack_table(t):
    n = t.shape[0]
    pairs = t.astype(jnp.bfloat16).reshape(n, _PK, 2)
    packed = lax.bitcast_convert_type(pairs, jnp.int32)
    return packed.reshape(n // 4, 4 * _PK)


def kernel(u, m, u_emb, m_emb, W1, b1, gamma, beta, W2, b2):
    ue, me = _build_gather()(u, m, _pack_table(u_emb), _pack_table(m_emb))
    w1p = jnp.take(W1, jnp.asarray(_PERM), axis=0)
    return _mlp(ue, me, u.reshape(B, 1), m.reshape(B, 1), w1p,
                b1.reshape(1, H), gamma.reshape(1, H), beta.reshape(1, H),
                W2.reshape(1, H), b2.reshape(1, 1))


# speedup vs baseline: 4.2964x; 4.2964x over previous
import functools
import math

import jax
import jax.numpy as jnp
from jax import lax
from jax.experimental import pallas as pl
from jax.experimental.pallas import tpu as pltpu
from jax.experimental.pallas import tpu_sc as plsc

B = 16384
D = 64
H = 1024
BN_EPS = 1e-5
_BN_INV = float(1.0 / math.sqrt(1.0 + BN_EPS))

_NC, _NS = 2, 16
_NW = _NC * _NS
_BPW = B // _NW
_PCH = 256
_PK = 32
_CB = 4096


def _repack_body(t_ref, out_ref):
    eye = (lax.broadcasted_iota(jnp.int32, (D, D), 0)
           == lax.broadcasted_iota(jnp.int32, (D, D), 1)).astype(jnp.bfloat16)
    q = _CB // 4
    for s in range(4):
        xb = t_ref[:, s * q:(s + 1) * q].astype(jnp.bfloat16)
        xt = lax.dot_general(xb, eye, (((0,), (0,)), ((), ())),
                             preferred_element_type=jnp.float32)
        lo = lax.bitcast_convert_type(xt[:, :_PK], jnp.int32)
        hi = lax.bitcast_convert_type(xt[:, _PK:], jnp.int32)
        w = ((lo >> 16) & jnp.int32(0xFFFF)) | (hi & jnp.int32(-65536))
        out_ref[:, s * _PK:(s + 1) * _PK] = w


def _make_repack(n):
    steps = -(-n // _CB)
    return pl.pallas_call(
        _repack_body,
        grid=(steps,),
        in_specs=[pl.BlockSpec((D, _CB), lambda i: (0, i))],
        out_specs=pl.BlockSpec((_CB // 4, 4 * _PK), lambda i: (i, 0)),
        out_shape=jax.ShapeDtypeStruct((steps * _CB // 4, 4 * _PK), jnp.int32),
    )


def _gather_body(u_hbm, m_hbm, ut_hbm, mt_hbm, ue_out, me_out,
                 uidx_v, midx_v, urows, mrows, sem_u, sem_m):
    wid = lax.axis_index("s") * _NC + lax.axis_index("c")
    base = wid * _BPW
    pltpu.sync_copy(u_hbm.at[pl.ds(base, _BPW)], uidx_v)
    pltpu.sync_copy(m_hbm.at[pl.ds(base, _BPW)], midx_v)

    for p in range(_BPW // _PCH):
        off = p * _PCH

        def _grp(g, carry):
            ur = uidx_v[pl.ds(off + g * 16, 16)]
            mr = midx_v[pl.ds(off + g * 16, 16)]
            uv = ((ur >> 12) << 10) | (ur & 1023)
            mv = ((mr >> 12) << 10) | (mr & 1023)
            for j in range(16):
                r = g * 16 + j
                pltpu.async_copy(ut_hbm.at[uv[j]], urows.at[r], sem_u)
                pltpu.async_copy(mt_hbm.at[mv[j]], mrows.at[r], sem_m)
            return carry

        lax.fori_loop(0, _PCH // 16, _grp, 0)
        pltpu.make_async_copy(ut_hbm.at[pl.ds(0, _PCH)], urows, sem_u).wait()
        pltpu.make_async_copy(mt_hbm.at[pl.ds(0, _PCH)], mrows, sem_m).wait()
        dst = pl.ds(base + off, _PCH)
        pltpu.sync_copy(urows, ue_out.at[dst])
        pltpu.sync_copy(mrows, me_out.at[dst])


@functools.cache
def _build_gather():
    mesh = plsc.VectorSubcoreMesh(core_axis_name="c", subcore_axis_name="s",
                                  num_cores=_NC, num_subcores=_NS)
    return pl.kernel(
        _gather_body,
        mesh=mesh,
        out_type=[jax.ShapeDtypeStruct((B, 4 * _PK), jnp.int32),
                  jax.ShapeDtypeStruct((B, 4 * _PK), jnp.int32)],
        scratch_types=[
            pltpu.VMEM((_BPW,), jnp.int32),
            pltpu.VMEM((_BPW,), jnp.int32),
            pltpu.VMEM((_PCH, 4 * _PK), jnp.int32),
            pltpu.VMEM((_PCH, 4 * _PK), jnp.int32),
            pltpu.SemaphoreType.DMA,
            pltpu.SemaphoreType.DMA,
        ],
    )


_BLK = 1024


def _unpack_quarter(rows, q):
    w01 = jnp.where(q == 0, rows[:, 0:_PK], rows[:, _PK:2 * _PK])
    w23 = jnp.where(q == 2, rows[:, 2 * _PK:3 * _PK], rows[:, 3 * _PK:])
    w = jnp.where(q < 2, w01, w23)
    lo = lax.bitcast_convert_type(w << 16, jnp.float32)
    hi = lax.bitcast_convert_type(w & jnp.int32(-65536), jnp.float32)
    return lo, hi


def _mlp_body(ue_ref, me_ref, u_ref, m_ref, w1_ref, b1_ref, gamma_ref,
              beta_ref, w2_ref, b2_ref, out_ref):
    ulo, uhi = _unpack_quarter(ue_ref[...], (u_ref[...] >> 10) & 3)
    mlo, mhi = _unpack_quarter(me_ref[...], (m_ref[...] >> 10) & 3)
    x = jnp.concatenate([ulo, uhi, mlo, mhi], axis=1)
    h = jnp.dot(x, w1_ref[...], preferred_element_type=jnp.float32)
    h = jnp.maximum(h + b1_ref[...], 0.0)
    h = h * (gamma_ref[...] * _BN_INV) + beta_ref[...]
    out = jnp.sum(h * w2_ref[...], axis=1, keepdims=True) + b2_ref[...]
    out_ref[...] = out


_mlp = pl.pallas_call(
    _mlp_body,
    grid=(B // _BLK,),
    in_specs=[
        pl.BlockSpec((_BLK, 4 * _PK), lambda i: (i, 0)),
        pl.BlockSpec((_BLK, 4 * _PK), lambda i: (i, 0)),
        pl.BlockSpec((_BLK, 1), lambda i: (i, 0)),
        pl.BlockSpec((_BLK, 1), lambda i: (i, 0)),
        pl.BlockSpec((2 * D, H), lambda i: (0, 0)),
        pl.BlockSpec((1, H), lambda i: (0, 0)),
        pl.BlockSpec((1, H), lambda i: (0, 0)),
        pl.BlockSpec((1, H), lambda i: (0, 0)),
        pl.BlockSpec((1, H), lambda i: (0, 0)),
        pl.BlockSpec((1, 1), lambda i: (0, 0)),
    ],
    out_specs=pl.BlockSpec((_BLK, 1), lambda i: (i, 0)),
    out_shape=jax.ShapeDtypeStruct((B, 1), jnp.float32),
)


def kernel(u, m, u_emb, m_emb, W1, b1, gamma, beta, W2, b2):
    up = _make_repack(u_emb.shape[0])(u_emb.T)
    mp = _make_repack(m_emb.shape[0])(m_emb.T)
    ue, me = _build_gather()(u, m, up, mp)
    return _mlp(ue, me, u.reshape(B, 1), m.reshape(B, 1), W1,
                b1.reshape(1, H), gamma.reshape(1, H), beta.reshape(1, H),
                W2.reshape(1, H), b2.reshape(1, 1))


# speedup vs baseline: 5.3102x; 1.2360x over previous
import functools
import math

import jax
import jax.numpy as jnp
from jax import lax
from jax.experimental import pallas as pl
from jax.experimental.pallas import tpu as pltpu
from jax.experimental.pallas import tpu_sc as plsc

B = 16384
D = 64
H = 1024
BN_EPS = 1e-5
_BN_INV = float(1.0 / math.sqrt(1.0 + BN_EPS))

_NC, _NS = 2, 16
_NW = _NC * _NS
_BPW = B // _NW
_PCH = 256
_PK = 32
_CB = 16384
_QL = 12


def _repack_body(t_ref, out_ref):
    eye = (lax.broadcasted_iota(jnp.int32, (D, D), 0)
           == lax.broadcasted_iota(jnp.int32, (D, D), 1)).astype(jnp.bfloat16)
    q = _CB // 4
    xb = t_ref[...].astype(jnp.bfloat16)
    xt = lax.dot_general(xb, eye, (((0,), (0,)), ((), ())),
                         preferred_element_type=jnp.float32)
    lo = lax.bitcast_convert_type(xt[:, :_PK], jnp.int32)
    hi = lax.bitcast_convert_type(xt[:, _PK:], jnp.int32)
    w = ((lo >> 16) & jnp.int32(0xFFFF)) | (hi & jnp.int32(-65536))
    for s in range(4):
        out_ref[:, s * _PK:(s + 1) * _PK] = w[s * q:(s + 1) * q, :]


def _make_repack(n):
    steps = -(-n // _CB)
    return pl.pallas_call(
        _repack_body,
        grid=(steps,),
        in_specs=[pl.BlockSpec((D, _CB), lambda i: (0, i))],
        out_specs=pl.BlockSpec((_CB // 4, 4 * _PK), lambda i: (i, 0)),
        out_shape=jax.ShapeDtypeStruct((steps * _CB // 4, 4 * _PK), jnp.int32),
    )


def _gather_body(u_hbm, m_hbm, ut_hbm, mt_hbm, ue_out, me_out,
                 uidx_v, midx_v, urows, mrows, sem_u, sem_m):
    wid = lax.axis_index("s") * _NC + lax.axis_index("c")
    base = wid * _BPW
    pltpu.sync_copy(u_hbm.at[pl.ds(base, _BPW)], uidx_v)
    pltpu.sync_copy(m_hbm.at[pl.ds(base, _BPW)], midx_v)

    for p in range(_BPW // _PCH):
        off = p * _PCH

        def _grp(g, carry):
            ur = uidx_v[pl.ds(off + g * 16, 16)]
            mr = midx_v[pl.ds(off + g * 16, 16)]
            uv = ((ur >> 14) << 12) | (ur & 4095)
            mv = ((mr >> 14) << 12) | (mr & 4095)
            for j in range(16):
                r = g * 16 + j
                pltpu.async_copy(ut_hbm.at[uv[j]], urows.at[r], sem_u)
                pltpu.async_copy(mt_hbm.at[mv[j]], mrows.at[r], sem_m)
            return carry

        lax.fori_loop(0, _PCH // 16, _grp, 0)
        pltpu.make_async_copy(ut_hbm.at[pl.ds(0, _PCH)], urows, sem_u).wait()
        pltpu.make_async_copy(mt_hbm.at[pl.ds(0, _PCH)], mrows, sem_m).wait()
        dst = pl.ds(base + off, _PCH)
        pltpu.sync_copy(urows, ue_out.at[dst])
        pltpu.sync_copy(mrows, me_out.at[dst])


@functools.cache
def _build_gather():
    mesh = plsc.VectorSubcoreMesh(core_axis_name="c", subcore_axis_name="s",
                                  num_cores=_NC, num_subcores=_NS)
    return pl.kernel(
        _gather_body,
        mesh=mesh,
        out_type=[jax.ShapeDtypeStruct((B, 4 * _PK), jnp.int32),
                  jax.ShapeDtypeStruct((B, 4 * _PK), jnp.int32)],
        scratch_types=[
            pltpu.VMEM((_BPW,), jnp.int32),
            pltpu.VMEM((_BPW,), jnp.int32),
            pltpu.VMEM((_PCH, 4 * _PK), jnp.int32),
            pltpu.VMEM((_PCH, 4 * _PK), jnp.int32),
            pltpu.SemaphoreType.DMA,
            pltpu.SemaphoreType.DMA,
        ],
    )


_BLK = 1024


def _unpack_quarter(rows, q):
    w01 = jnp.where(q == 0, rows[:, 0:_PK], rows[:, _PK:2 * _PK])
    w23 = jnp.where(q == 2, rows[:, 2 * _PK:3 * _PK], rows[:, 3 * _PK:])
    w = jnp.where(q < 2, w01, w23)
    lo = lax.bitcast_convert_type(w << 16, jnp.float32)
    hi = lax.bitcast_convert_type(w & jnp.int32(-65536), jnp.float32)
    return lo, hi


def _mlp_body(ue_ref, me_ref, u_ref, m_ref, w1_ref, b1_ref, gamma_ref,
              beta_ref, w2_ref, b2_ref, out_ref):
    ulo, uhi = _unpack_quarter(ue_ref[...], (u_ref[...] >> _QL) & 3)
    mlo, mhi = _unpack_quarter(me_ref[...], (m_ref[...] >> _QL) & 3)
    x = jnp.concatenate([ulo, uhi, mlo, mhi], axis=1)
    h = jnp.dot(x, w1_ref[...], preferred_element_type=jnp.float32)
    h = jnp.maximum(h + b1_ref[...], 0.0)
    h = h * (gamma_ref[...] * _BN_INV) + beta_ref[...]
    out = jnp.sum(h * w2_ref[...], axis=1, keepdims=True) + b2_ref[...]
    out_ref[...] = out


_mlp = pl.pallas_call(
    _mlp_body,
    grid=(B // _BLK,),
    in_specs=[
        pl.BlockSpec((_BLK, 4 * _PK), lambda i: (i, 0)),
        pl.BlockSpec((_BLK, 4 * _PK), lambda i: (i, 0)),
        pl.BlockSpec((_BLK, 1), lambda i: (i, 0)),
        pl.BlockSpec((_BLK, 1), lambda i: (i, 0)),
        pl.BlockSpec((2 * D, H), lambda i: (0, 0)),
        pl.BlockSpec((1, H), lambda i: (0, 0)),
        pl.BlockSpec((1, H), lambda i: (0, 0)),
        pl.BlockSpec((1, H), lambda i: (0, 0)),
        pl.BlockSpec((1, H), lambda i: (0, 0)),
        pl.BlockSpec((1, 1), lambda i: (0, 0)),
    ],
    out_specs=pl.BlockSpec((_BLK, 1), lambda i: (i, 0)),
    out_shape=jax.ShapeDtypeStruct((B, 1), jnp.float32),
)


def kernel(u, m, u_emb, m_emb, W1, b1, gamma, beta, W2, b2):
    up = _make_repack(u_emb.shape[0])(u_emb.T)
    mp = _make_repack(m_emb.shape[0])(m_emb.T)
    ue, me = _build_gather()(u, m, up, mp)
    return _mlp(ue, me, u.reshape(B, 1), m.reshape(B, 1), W1,
                b1.reshape(1, H), gamma.reshape(1, H), beta.reshape(1, H),
                W2.reshape(1, H), b2.reshape(1, 1))


# speedup vs baseline: 5.5526x; 1.0456x over previous
import functools
import math

import jax
import jax.numpy as jnp
from jax import lax
from jax.experimental import pallas as pl
from jax.experimental.pallas import tpu as pltpu
from jax.experimental.pallas import tpu_sc as plsc

B = 16384
D = 64
H = 1024
BN_EPS = 1e-5
_BN_INV = float(1.0 / math.sqrt(1.0 + BN_EPS))

_NC, _NS = 2, 16
_NW = _NC * _NS
_BPW = B // _NW
_PCH = 256
_PK = 32
_CB = 16384
_QL = 12


def _repack_body(t_ref, out_ref):
    q = _CB // 4
    ub = lax.bitcast_convert_type(t_ref[...], jnp.uint32)
    w_pre = (ub[:_PK, :] >> 16) | (ub[_PK:, :] & jnp.uint32(0xFFFF0000))
    w = lax.bitcast_convert_type(jnp.transpose(w_pre), jnp.int32)
    for s in range(4):
        out_ref[:, s * _PK:(s + 1) * _PK] = w[s * q:(s + 1) * q, :]


def _make_repack(n):
    steps = -(-n // _CB)
    return pl.pallas_call(
        _repack_body,
        grid=(steps,),
        in_specs=[pl.BlockSpec((D, _CB), lambda i: (0, i))],
        out_specs=pl.BlockSpec((_CB // 4, 4 * _PK), lambda i: (i, 0)),
        out_shape=jax.ShapeDtypeStruct((steps * _CB // 4, 4 * _PK), jnp.int32),
    )


def _gather_body(u_hbm, m_hbm, ut_hbm, mt_hbm, ue_out, me_out,
                 uidx_v, midx_v, urows, mrows, sem_u, sem_m):
    wid = lax.axis_index("s") * _NC + lax.axis_index("c")
    base = wid * _BPW
    pltpu.sync_copy(u_hbm.at[pl.ds(base, _BPW)], uidx_v)
    pltpu.sync_copy(m_hbm.at[pl.ds(base, _BPW)], midx_v)

    for p in range(_BPW // _PCH):
        off = p * _PCH

        def _grp(g, carry):
            ur = uidx_v[pl.ds(off + g * 16, 16)]
            mr = midx_v[pl.ds(off + g * 16, 16)]
            uv = ((ur >> 14) << 12) | (ur & 4095)
            mv = ((mr >> 14) << 12) | (mr & 4095)
            for j in range(16):
                r = g * 16 + j
                pltpu.async_copy(ut_hbm.at[uv[j]], urows.at[r], sem_u)
                pltpu.async_copy(mt_hbm.at[mv[j]], mrows.at[r], sem_m)
            return carry

        lax.fori_loop(0, _PCH // 16, _grp, 0)
        pltpu.make_async_copy(ut_hbm.at[pl.ds(0, _PCH)], urows, sem_u).wait()
        pltpu.make_async_copy(mt_hbm.at[pl.ds(0, _PCH)], mrows, sem_m).wait()
        dst = pl.ds(base + off, _PCH)
        pltpu.sync_copy(urows, ue_out.at[dst])
        pltpu.sync_copy(mrows, me_out.at[dst])


@functools.cache
def _build_gather():
    mesh = plsc.VectorSubcoreMesh(core_axis_name="c", subcore_axis_name="s",
                                  num_cores=_NC, num_subcores=_NS)
    return pl.kernel(
        _gather_body,
        mesh=mesh,
        out_type=[jax.ShapeDtypeStruct((B, 4 * _PK), jnp.int32),
                  jax.ShapeDtypeStruct((B, 4 * _PK), jnp.int32)],
        scratch_types=[
            pltpu.VMEM((_BPW,), jnp.int32),
            pltpu.VMEM((_BPW,), jnp.int32),
            pltpu.VMEM((_PCH, 4 * _PK), jnp.int32),
            pltpu.VMEM((_PCH, 4 * _PK), jnp.int32),
            pltpu.SemaphoreType.DMA,
            pltpu.SemaphoreType.DMA,
        ],
    )


_BLK = 1024


def _unpack_quarter(rows, q):
    w01 = jnp.where(q == 0, rows[:, 0:_PK], rows[:, _PK:2 * _PK])
    w23 = jnp.where(q == 2, rows[:, 2 * _PK:3 * _PK], rows[:, 3 * _PK:])
    w = jnp.where(q < 2, w01, w23)
    lo = lax.bitcast_convert_type(w << 16, jnp.float32)
    hi = lax.bitcast_convert_type(w & jnp.int32(-65536), jnp.float32)
    return lo, hi


def _mlp_body(ue_ref, me_ref, u_ref, m_ref, w1_ref, b1_ref,
              w2p_ref, c_ref, out_ref):
    ulo, uhi = _unpack_quarter(ue_ref[...], (u_ref[...] >> _QL) & 3)
    mlo, mhi = _unpack_quarter(me_ref[...], (m_ref[...] >> _QL) & 3)
    x = jnp.concatenate([ulo, uhi, mlo, mhi], axis=1)
    h = jnp.dot(x, w1_ref[...], preferred_element_type=jnp.float32)
    h = jnp.maximum(h + b1_ref[...], 0.0)
    out = jnp.dot(h, w2p_ref[...], preferred_element_type=jnp.float32)
    out_ref[...] = out + c_ref[...]


_mlp = pl.pallas_call(
    _mlp_body,
    grid=(B // _BLK,),
    in_specs=[
        pl.BlockSpec((_BLK, 4 * _PK), lambda i: (i, 0)),
        pl.BlockSpec((_BLK, 4 * _PK), lambda i: (i, 0)),
        pl.BlockSpec((_BLK, 1), lambda i: (i, 0)),
        pl.BlockSpec((_BLK, 1), lambda i: (i, 0)),
        pl.BlockSpec((2 * D, H), lambda i: (0, 0)),
        pl.BlockSpec((1, H), lambda i: (0, 0)),
        pl.BlockSpec((H, 1), lambda i: (0, 0)),
        pl.BlockSpec((1, 1), lambda i: (0, 0)),
    ],
    out_specs=pl.BlockSpec((_BLK, 1), lambda i: (i, 0)),
    out_shape=jax.ShapeDtypeStruct((B, 1), jnp.float32),
)


def kernel(u, m, u_emb, m_emb, W1, b1, gamma, beta, W2, b2):
    up = _make_repack(u_emb.shape[0])(u_emb.T)
    mp = _make_repack(m_emb.shape[0])(m_emb.T)
    ue, me = _build_gather()(u, m, up, mp)
    scale = gamma * _BN_INV
    w2p = scale[:, None] * W2
    c = (beta @ W2 + b2).reshape(1, 1)
    return _mlp(ue, me, u.reshape(B, 1), m.reshape(B, 1), W1,
                b1.reshape(1, H), w2p, c)
